# baseline (speedup 1.0000x reference)
import functools

import jax
import jax.numpy as jnp
from jax import lax
from jax.experimental import pallas as pl
from jax.experimental.pallas import tpu as pltpu
from jax.experimental.pallas import tpu_sc as plsc

N = 10000
E = 320000
D = 128
H = 256
A = 16
G = 64
NC = 2
NS = 16
HB = H // 2

NP = N + 16
RPT = 624
TAIL = N - NS * RPT
TAIL_OFF = NS * RPT
EPT = E // NS
BLK = 16
NBLK = 10
CHP = NBLK * BLK
EPT_P = CHP * 128
EPD = E // (NC * NS)
DEG_CHUNKS, DEG_REM = EPD // 128, EPD % 128


def _mesh():
    return plsc.VectorSubcoreMesh(
        core_axis_name="c", subcore_axis_name="s", num_cores=NC, num_subcores=NS
    )


def _sc_deg_body(dst_hbm, degp_hbm, acc, dbuf, dbuf16, onesbuf, zbuf):
    c = lax.axis_index("c")
    s = lax.axis_index("s")

    def fill(i, _):
        onesbuf[i, pl.ds(0, 16)] = jnp.full((16,), 1.0, jnp.float32)
        return 0

    lax.fori_loop(0, 128, fill, 0)

    def zfill(i, _):
        zbuf[i, pl.ds(0, 16)] = jnp.zeros((16,), jnp.float32)
        return 0

    lax.fori_loop(0, RPT, zfill, 0)
    pltpu.sync_copy(zbuf, acc.at[pl.ds(s * RPT, RPT)])

    @pl.when(s == NS - 1)
    def _():
        pltpu.sync_copy(zbuf.at[pl.ds(0, TAIL)], acc.at[pl.ds(TAIL_OFF, TAIL)])

    plsc.subcore_barrier()

    base0 = c * (E // NC) + s * EPD

    def step(i, _):
        b = base0 + i * 128
        pltpu.sync_copy(dst_hbm.at[pl.ds(b, 128)], dbuf.at[0])
        pltpu.sync_copy(onesbuf, acc.at[dbuf.at[0]], add=True)
        return 0

    lax.fori_loop(0, DEG_CHUNKS, step, 0)
    b = base0 + DEG_CHUNKS * 128
    pltpu.sync_copy(dst_hbm.at[pl.ds(b, DEG_REM)], dbuf16.at[0])
    pltpu.sync_copy(onesbuf.at[pl.ds(0, DEG_REM)], acc.at[dbuf16.at[0]], add=True)
    plsc.subcore_barrier()
    pltpu.sync_copy(acc.at[pl.ds(s * RPT, RPT)], degp_hbm.at[c, pl.ds(s * RPT, RPT)])

    @pl.when(s == NS - 1)
    def _():
        pltpu.sync_copy(acc.at[pl.ds(TAIL_OFF, TAIL)],
                        degp_hbm.at[c, pl.ds(TAIL_OFF, TAIL)])


_sc_deg = functools.partial(
    pl.kernel,
    out_type=jax.ShapeDtypeStruct((NC, N, 16), jnp.float32),
    mesh=_mesh(),
    scratch_types=[
        pltpu.VMEM_SHARED((N, 16), jnp.float32),
        pltpu.VMEM((1, 128), jnp.int32),
        pltpu.VMEM((1, DEG_REM), jnp.int32),
        pltpu.VMEM((128, 16), jnp.float32),
        pltpu.VMEM((RPT, 16), jnp.float32),
    ],
)(_sc_deg_body)


def _sc_msg_body(y_hbm, src_hbm, dst_hbm, out_hbm, acc, sidx0, sidx1,
                 didx0, didx1, rowbuf0, rowbuf1, gsem0, gsem1, bsem0, bsem1):
    c = lax.axis_index("c")
    s = lax.axis_index("s")
    sidx = (sidx0, sidx1)
    didx = (didx0, didx1)
    rowbuf = (rowbuf0, rowbuf1)
    gsem = (gsem0, gsem1)
    bsem = (bsem0, bsem1)

    def start_blk(bb, p):
        pltpu.async_copy(src_hbm.at[c, s, bb], sidx[p], bsem[p])
        pltpu.async_copy(dst_hbm.at[s, bb], didx[p], bsem[p])

    def drain_blk(p):
        pltpu.make_async_copy(src_hbm.at[0, 0, 0], sidx[p], bsem[p]).wait()
        pltpu.make_async_copy(dst_hbm.at[0, 0], didx[p], bsem[p]).wait()

    def start_gather(p, bp, k):
        pltpu.async_copy(y_hbm.at[sidx[bp].at[k]], rowbuf[p], gsem[p])

    def drain_gather(p):
        pltpu.make_async_copy(y_hbm.at[pl.ds(0, 128)], rowbuf[p], gsem[p]).wait()

    def scatter(p, bp, k):
        pltpu.sync_copy(rowbuf[p], acc.at[didx[bp].at[k]], add=True)

    start_blk(0, 0)
    start_blk(1, 1)
    drain_blk(0)
    start_gather(0, 0, 0)

    def zrow(i, _):
        for j in range(8):
            rowbuf1[i, pl.ds(j * 16, 16)] = jnp.zeros((16,), jnp.float32)
        return 0

    lax.fori_loop(0, 128, zrow, 0)
    for k in range(4):
        pltpu.sync_copy(rowbuf1, acc.at[pl.ds(s * RPT + k * 128, 128)])
    pltpu.sync_copy(rowbuf1.at[pl.ds(0, RPT - 512)],
                    acc.at[pl.ds(s * RPT + 512, RPT - 512)])

    @pl.when(s == NS - 1)
    def _():
        pltpu.sync_copy(rowbuf1.at[pl.ds(0, TAIL)], acc.at[pl.ds(TAIL_OFF, TAIL)])

    plsc.subcore_barrier()

    NSB = NBLK // 2

    def halfblock(bp):
        def pairk(jj, _):
            k = jj * 2
            start_gather(1, bp, k + 1)
            drain_gather(0)
            scatter(0, bp, k)
            start_gather(0, bp, k + 2)
            drain_gather(1)
            scatter(1, bp, k + 1)
            return 0

        lax.fori_loop(0, BLK // 2 - 1, pairk, 0)

    def sblock(sb, _):
        drain_blk(1)
        halfblock(0)
        start_gather(1, 0, BLK - 1)
        drain_gather(0)
        scatter(0, 0, BLK - 2)
        start_gather(0, 1, 0)
        drain_gather(1)
        scatter(1, 0, BLK - 1)
        @pl.when(sb < NSB - 1)
        def _():
            start_blk(2 * sb + 2, 0)

        halfblock(1)

        @pl.when(sb < NSB - 1)
        def _():
            drain_blk(0)

        start_gather(1, 1, BLK - 1)
        drain_gather(0)
        scatter(0, 1, BLK - 2)

        @pl.when(sb < NSB - 1)
        def _():
            start_gather(0, 0, 0)

        drain_gather(1)
        scatter(1, 1, BLK - 1)

        @pl.when(sb < NSB - 1)
        def _():
            start_blk(2 * sb + 3, 1)

        return 0

    lax.fori_loop(0, NSB, sblock, 0)

    plsc.subcore_barrier()
    pltpu.sync_copy(acc.at[pl.ds(s * RPT, RPT)],
                    out_hbm.at[pl.ds(c * N + s * RPT, RPT)])

    @pl.when(s == NS - 1)
    def _():
        pltpu.sync_copy(acc.at[pl.ds(TAIL_OFF, TAIL)],
                        out_hbm.at[pl.ds(c * N + TAIL_OFF, TAIL)])


_sc_msg = functools.partial(
    pl.kernel,
    out_type=jax.ShapeDtypeStruct((NC * N, HB), jnp.float32),
    mesh=_mesh(),
    scratch_types=[
        pltpu.VMEM_SHARED((N, HB), jnp.float32),
        pltpu.VMEM((BLK, 128), jnp.int32),
        pltpu.VMEM((BLK, 128), jnp.int32),
        pltpu.VMEM((BLK, 128), jnp.int32),
        pltpu.VMEM((BLK, 128), jnp.int32),
        pltpu.VMEM((128, HB), jnp.float32),
        pltpu.VMEM((128, HB), jnp.float32),
        pltpu.SemaphoreType.DMA,
        pltpu.SemaphoreType.DMA,
        pltpu.SemaphoreType.DMA,
        pltpu.SemaphoreType.DMA,
    ],
)(_sc_msg_body)


def _tc_prep_body(x_ref, w1_ref, degp_ref, y_ref, dinv_ref):
    dp = degp_ref[...]
    deg = dp[0, :, 0:1] + dp[1, :, 0:1] + 1.0
    dinv = lax.rsqrt(deg)
    y = dinv * jnp.dot(x_ref[...], w1_ref[...], preferred_element_type=jnp.float32)
    y_ref[0, :N] = y[:, :HB]
    y_ref[0, N:] = jnp.zeros((NP - N, HB), jnp.float32)
    y_ref[1, :N] = y[:, HB:]
    y_ref[1, N:] = jnp.zeros((NP - N, HB), jnp.float32)
    dinv_ref[...] = dinv


_tc_prep = pl.pallas_call(
    _tc_prep_body,
    out_shape=(
        jax.ShapeDtypeStruct((NC, NP, HB), jnp.float32),
        jax.ShapeDtypeStruct((N, 1), jnp.float32),
    ),
)


def _tc_mid_body(m_ref, y_ref, dinv_ref, b_ref, g_ref, be_ref, w_ref, o_ref):
    dinv = dinv_ref[...]
    t = jnp.concatenate([m_ref[0] + y_ref[0, :N], m_ref[1] + y_ref[1, :N]],
                        axis=1)
    t = jax.nn.relu(dinv * t + b_ref[...])
    mu = jnp.mean(t, axis=0, keepdims=True)
    var = jnp.mean((t - mu) ** 2, axis=0, keepdims=True)
    h = (t - mu) * lax.rsqrt(var + 1e-5) * g_ref[...] + be_ref[...]
    yn = dinv * jnp.dot(h, w_ref[...], preferred_element_type=jnp.float32)
    o_ref[0, :N] = yn[:, :HB]
    o_ref[0, N:] = jnp.zeros((NP - N, HB), jnp.float32)
    o_ref[1, :N] = yn[:, HB:]
    o_ref[1, N:] = jnp.zeros((NP - N, HB), jnp.float32)


_tc_mid = pl.pallas_call(
    _tc_mid_body,
    out_shape=jax.ShapeDtypeStruct((NC, NP, HB), jnp.float32),
)


def _tc_head_body(m_ref, y_ref, dinv_ref, b_ref, bt_ref, act_ref, wa_ref,
                  wb_ref, b1_ref, w2_ref, b2_ref, w3_ref, b3_ref, g_ref,
                  be_ref, o_ref):
    dinv = dinv_ref[...]
    t = jnp.concatenate([m_ref[0] + y_ref[0, :N], m_ref[1] + y_ref[1, :N]],
                        axis=1)
    h = jax.nn.relu(dinv * t + b_ref[...])
    bt = bt_ref[...]
    gi = lax.broadcasted_iota(jnp.int32, (G, N), 0)
    mt = (gi == bt).astype(jnp.float32)
    ssum = jnp.dot(mt, h, preferred_element_type=jnp.float32)
    cnt = jnp.sum(mt, axis=1, keepdims=True)
    pooled = ssum / jnp.maximum(cnt, 1.0)
    z = (jnp.dot(pooled, wa_ref[...], preferred_element_type=jnp.float32)
         + jnp.dot(act_ref[...], wb_ref[...], preferred_element_type=jnp.float32)
         + b1_ref[...])
    z = jax.nn.relu(z)
    mu = jnp.mean(z, axis=0, keepdims=True)
    var = jnp.mean((z - mu) ** 2, axis=0, keepdims=True)
    z = (z - mu) * lax.rsqrt(var + 1e-5) * g_ref[...] + be_ref[...]
    z = jax.nn.relu(jnp.dot(z, w2_ref[...], preferred_element_type=jnp.float32)
                    + b2_ref[...])
    o_ref[...] = jnp.dot(z, w3_ref[...], preferred_element_type=jnp.float32) + b3_ref[...]


_tc_head = pl.pallas_call(
    _tc_head_body,
    out_shape=jax.ShapeDtypeStruct((G, 1), jnp.float32),
)


def kernel(x, edge_index, batch, actions, W1, b1, W2, b2, W3, b3, g1, be1,
           g2, be2, g3, be3, fc1_W, fc1_b, fc2_W, fc2_b, fc3_W, fc3_b):
    src = edge_index[0]
    dst = edge_index[1]
    srcp = jnp.pad(src.reshape(NS, EPT), ((0, 0), (0, EPT_P - EPT)),
                   constant_values=N).reshape(NS, NBLK, BLK, 128)
    src4 = jnp.stack([srcp, srcp + NP])
    dst3 = jnp.pad(dst.reshape(NS, EPT), ((0, 0), (0, EPT_P - EPT)),
                   constant_values=0).reshape(NS, NBLK, BLK, 128)
    degp = _sc_deg(dst)
    y1, dinv = _tc_prep(x, W1, degp)
    m1 = _sc_msg(y1.reshape(NC * NP, HB), src4, dst3).reshape(NC, N, HB)
    y2 = _tc_mid(m1, y1, dinv, b1.reshape(1, H), g1.reshape(1, H),
                 be1.reshape(1, H), W2)
    m2 = _sc_msg(y2.reshape(NC * NP, HB), src4, dst3).reshape(NC, N, HB)
    y3 = _tc_mid(m2, y2, dinv, b2.reshape(1, H), g2.reshape(1, H),
                 be2.reshape(1, H), W3)
    m3 = _sc_msg(y3.reshape(NC * NP, HB), src4, dst3).reshape(NC, N, HB)
    return _tc_head(m3, y3, dinv, b3.reshape(1, H), batch.reshape(1, N),
                    actions.reshape(1, A), fc1_W[:H], fc1_W[H:],
                    fc1_b.reshape(1, H), fc2_W, fc2_b.reshape(1, H // 2),
                    fc3_W, fc3_b.reshape(1, 1), g3.reshape(1, H),
                    be3.reshape(1, H))


# speedup vs baseline: 1.5197x; 1.5197x over previous
import functools

import jax
import jax.numpy as jnp
from jax import lax
from jax.experimental import pallas as pl
from jax.experimental.pallas import tpu as pltpu
from jax.experimental.pallas import tpu_sc as plsc

N = 10000
E = 320000
D = 128
H = 256
A = 16
G = 64
NC = 2
NS = 16
HB = H // 2

NP = N + 16
RPT = 624
TAIL = N - NS * RPT
TAIL_OFF = NS * RPT
EPT = E // NS
CHP = 158
EPT_P = CHP * 128
EPD = E // (NC * NS)
DEG_CHUNKS, DEG_REM = EPD // 128, EPD % 128


def _mesh():
    return plsc.VectorSubcoreMesh(
        core_axis_name="c", subcore_axis_name="s", num_cores=NC, num_subcores=NS
    )


def _sc_deg_body(dst_hbm, degp_hbm, acc, dbuf, dbuf16, onesbuf, zbuf):
    c = lax.axis_index("c")
    s = lax.axis_index("s")

    def fill(i, _):
        onesbuf[i, pl.ds(0, 16)] = jnp.full((16,), 1.0, jnp.float32)
        return 0

    lax.fori_loop(0, 128, fill, 0)

    def zfill(i, _):
        zbuf[i, pl.ds(0, 16)] = jnp.zeros((16,), jnp.float32)
        return 0

    lax.fori_loop(0, RPT, zfill, 0)
    pltpu.sync_copy(zbuf, acc.at[pl.ds(s * RPT, RPT)])

    @pl.when(s == NS - 1)
    def _():
        pltpu.sync_copy(zbuf.at[pl.ds(0, TAIL)], acc.at[pl.ds(TAIL_OFF, TAIL)])

    plsc.subcore_barrier()

    base0 = c * (E // NC) + s * EPD

    def step(i, _):
        b = base0 + i * 128
        pltpu.sync_copy(dst_hbm.at[pl.ds(b, 128)], dbuf.at[0])
        pltpu.sync_copy(onesbuf, acc.at[dbuf.at[0]], add=True)
        return 0

    lax.fori_loop(0, DEG_CHUNKS, step, 0)
    b = base0 + DEG_CHUNKS * 128
    pltpu.sync_copy(dst_hbm.at[pl.ds(b, DEG_REM)], dbuf16.at[0])
    pltpu.sync_copy(onesbuf.at[pl.ds(0, DEG_REM)], acc.at[dbuf16.at[0]], add=True)
    plsc.subcore_barrier()
    pltpu.sync_copy(acc.at[pl.ds(s * RPT, RPT)], degp_hbm.at[c, pl.ds(s * RPT, RPT)])

    @pl.when(s == NS - 1)
    def _():
        pltpu.sync_copy(acc.at[pl.ds(TAIL_OFF, TAIL)],
                        degp_hbm.at[c, pl.ds(TAIL_OFF, TAIL)])


_sc_deg = functools.partial(
    pl.kernel,
    out_type=jax.ShapeDtypeStruct((NC, N, 16), jnp.float32),
    mesh=_mesh(),
    scratch_types=[
        pltpu.VMEM_SHARED((N, 16), jnp.float32),
        pltpu.VMEM((1, 128), jnp.int32),
        pltpu.VMEM((1, DEG_REM), jnp.int32),
        pltpu.VMEM((128, 16), jnp.float32),
        pltpu.VMEM((RPT, 16), jnp.float32),
    ],
)(_sc_deg_body)


def _sc_msg_body(y_hbm, src_hbm, dst_hbm, out_hbm, acc, sbuf0, sbuf1, sbuf2,
                 dbuf0, dbuf1, dbuf2, rowbuf0, rowbuf1, rowbuf2,
                 gsem0, gsem1, gsem2, ssem0, ssem1, ssem2,
                 sisem0, sisem1, sisem2, disem0, disem1, disem2):
    c = lax.axis_index("c")
    s = lax.axis_index("s")
    sbuf = (sbuf0, sbuf1, sbuf2)
    dbuf = (dbuf0, dbuf1, dbuf2)
    rowbuf = (rowbuf0, rowbuf1, rowbuf2)
    gsem = (gsem0, gsem1, gsem2)
    ssem = (ssem0, ssem1, ssem2)
    sisem = (sisem0, sisem1, sisem2)
    disem = (disem0, disem1, disem2)

    def start_sidx(i, b):
        pltpu.async_copy(src_hbm.at[c, s, i], sbuf[b], sisem[b])

    def drain_sidx(b):
        pltpu.make_async_copy(src_hbm.at[0, 0, 0], sbuf[b], sisem[b]).wait()

    def start_didx(i, b):
        pltpu.async_copy(dst_hbm.at[s, i], dbuf[b].at[0], disem[b])

    def drain_didx(b):
        pltpu.make_async_copy(dst_hbm.at[0, 0], dbuf[b].at[0], disem[b]).wait()

    def start_gather(b):
        pltpu.async_copy(y_hbm.at[sbuf[b]], rowbuf[b], gsem[b])

    def drain_gather(b):
        pltpu.make_async_copy(y_hbm.at[pl.ds(0, 128)], rowbuf[b], gsem[b]).wait()

    def start_scatter(b):
        pltpu.async_copy(rowbuf[b], acc.at[dbuf[b].at[0]], ssem[b], add=True)

    def drain_scatter(b):
        pltpu.make_async_copy(rowbuf[b], acc.at[pl.ds(0, 128)], ssem[b]).wait()

    def slot(i, b, skip_first_drain=False, last=CHP - 1):
        b2 = (b + 2) % 3
        if not skip_first_drain:
            drain_scatter(b2)
        if not isinstance(i, int) or i + 2 <= last:
            drain_sidx(b2)
            start_gather(b2)
            start_didx(i + 2, b2)
        drain_gather(b)
        if not isinstance(i, int) or i + 3 <= last:
            start_sidx(i + 3, b)
        drain_didx(b)
        start_scatter(b)

    for b in range(3):
        start_sidx(b, b)
    start_didx(0, 0)
    start_didx(1, 1)
    drain_sidx(0)
    start_gather(0)
    drain_sidx(1)
    start_gather(1)

    def zrow(i, _):
        for j in range(8):
            rowbuf2[i, pl.ds(j * 16, 16)] = jnp.zeros((16,), jnp.float32)
        return 0

    lax.fori_loop(0, 128, zrow, 0)
    for k in range(4):
        pltpu.sync_copy(rowbuf2, acc.at[pl.ds(s * RPT + k * 128, 128)])
    pltpu.sync_copy(rowbuf2.at[pl.ds(0, RPT - 512)],
                    acc.at[pl.ds(s * RPT + 512, RPT - 512)])

    @pl.when(s == NS - 1)
    def _():
        pltpu.sync_copy(rowbuf2.at[pl.ds(0, TAIL)], acc.at[pl.ds(TAIL_OFF, TAIL)])

    plsc.subcore_barrier()

    slot(0, 0, skip_first_drain=True)

    def tri(jj, _):
        i = 1 + jj * 3
        slot(i, 1)
        slot(i + 1, 2)
        slot(i + 2, 0)
        return 0

    lax.fori_loop(0, 51, tri, 0)

    for i in range(154, CHP):
        slot(i, i % 3)
    drain_scatter((CHP - 1) % 3)

    plsc.subcore_barrier()
    pltpu.sync_copy(acc.at[pl.ds(s * RPT, RPT)],
                    out_hbm.at[pl.ds(c * N + s * RPT, RPT)])

    @pl.when(s == NS - 1)
    def _():
        pltpu.sync_copy(acc.at[pl.ds(TAIL_OFF, TAIL)],
                        out_hbm.at[pl.ds(c * N + TAIL_OFF, TAIL)])


_sc_msg = functools.partial(
    pl.kernel,
    out_type=jax.ShapeDtypeStruct((NC * N, HB), jnp.float32),
    mesh=_mesh(),
    scratch_types=(
        [pltpu.VMEM_SHARED((N, HB), jnp.float32)]
        + [pltpu.VMEM((128,), jnp.int32)] * 3
        + [pltpu.VMEM((1, 128), jnp.int32)] * 3
        + [pltpu.VMEM((128, HB), jnp.float32)] * 3
        + [pltpu.SemaphoreType.DMA] * 12
    ),
)(_sc_msg_body)


def _tc_prep_body(x_ref, w1_ref, degp_ref, y_ref, dinv_ref):
    dp = degp_ref[...]
    deg = dp[0, :, 0:1] + dp[1, :, 0:1] + 1.0
    dinv = lax.rsqrt(deg)
    y = dinv * jnp.dot(x_ref[...], w1_ref[...], preferred_element_type=jnp.float32)
    y_ref[0, :N] = y[:, :HB]
    y_ref[0, N:] = jnp.zeros((NP - N, HB), jnp.float32)
    y_ref[1, :N] = y[:, HB:]
    y_ref[1, N:] = jnp.zeros((NP - N, HB), jnp.float32)
    dinv_ref[...] = dinv


_tc_prep = pl.pallas_call(
    _tc_prep_body,
    out_shape=(
        jax.ShapeDtypeStruct((NC, NP, HB), jnp.float32),
        jax.ShapeDtypeStruct((N, 1), jnp.float32),
    ),
)


def _tc_mid_body(m_ref, y_ref, dinv_ref, b_ref, g_ref, be_ref, w_ref, o_ref):
    dinv = dinv_ref[...]
    t = jnp.concatenate([m_ref[0] + y_ref[0, :N], m_ref[1] + y_ref[1, :N]],
                        axis=1)
    t = jax.nn.relu(dinv * t + b_ref[...])
    mu = jnp.mean(t, axis=0, keepdims=True)
    var = jnp.mean((t - mu) ** 2, axis=0, keepdims=True)
    h = (t - mu) * lax.rsqrt(var + 1e-5) * g_ref[...] + be_ref[...]
    yn = dinv * jnp.dot(h, w_ref[...], preferred_element_type=jnp.float32)
    o_ref[0, :N] = yn[:, :HB]
    o_ref[0, N:] = jnp.zeros((NP - N, HB), jnp.float32)
    o_ref[1, :N] = yn[:, HB:]
    o_ref[1, N:] = jnp.zeros((NP - N, HB), jnp.float32)


_tc_mid = pl.pallas_call(
    _tc_mid_body,
    out_shape=jax.ShapeDtypeStruct((NC, NP, HB), jnp.float32),
)


def _tc_head_body(m_ref, y_ref, dinv_ref, b_ref, bt_ref, act_ref, wa_ref,
                  wb_ref, b1_ref, w2_ref, b2_ref, w3_ref, b3_ref, g_ref,
                  be_ref, o_ref):
    dinv = dinv_ref[...]
    t = jnp.concatenate([m_ref[0] + y_ref[0, :N], m_ref[1] + y_ref[1, :N]],
                        axis=1)
    h = jax.nn.relu(dinv * t + b_ref[...])
    bt = bt_ref[...]
    gi = lax.broadcasted_iota(jnp.int32, (G, N), 0)
    mt = (gi == bt).astype(jnp.float32)
    ssum = jnp.dot(mt, h, preferred_element_type=jnp.float32)
    cnt = jnp.sum(mt, axis=1, keepdims=True)
    pooled = ssum / jnp.maximum(cnt, 1.0)
    z = (jnp.dot(pooled, wa_ref[...], preferred_element_type=jnp.float32)
         + jnp.dot(act_ref[...], wb_ref[...], preferred_element_type=jnp.float32)
         + b1_ref[...])
    z = jax.nn.relu(z)
    mu = jnp.mean(z, axis=0, keepdims=True)
    var = jnp.mean((z - mu) ** 2, axis=0, keepdims=True)
    z = (z - mu) * lax.rsqrt(var + 1e-5) * g_ref[...] + be_ref[...]
    z = jax.nn.relu(jnp.dot(z, w2_ref[...], preferred_element_type=jnp.float32)
                    + b2_ref[...])
    o_ref[...] = jnp.dot(z, w3_ref[...], preferred_element_type=jnp.float32) + b3_ref[...]


_tc_head = pl.pallas_call(
    _tc_head_body,
    out_shape=jax.ShapeDtypeStruct((G, 1), jnp.float32),
)


def kernel(x, edge_index, batch, actions, W1, b1, W2, b2, W3, b3, g1, be1,
           g2, be2, g3, be3, fc1_W, fc1_b, fc2_W, fc2_b, fc3_W, fc3_b):
    src = edge_index[0]
    dst = edge_index[1]
    srcp = jnp.pad(src.reshape(NS, EPT), ((0, 0), (0, EPT_P - EPT)),
                   constant_values=N).reshape(NS, CHP, 128)
    src4 = jnp.stack([srcp, srcp + NP])
    dst3 = jnp.pad(dst.reshape(NS, EPT), ((0, 0), (0, EPT_P - EPT)),
                   constant_values=0).reshape(NS, CHP, 128)
    degp = _sc_deg(dst)
    y1, dinv = _tc_prep(x, W1, degp)
    m1 = _sc_msg(y1.reshape(NC * NP, HB), src4, dst3).reshape(NC, N, HB)
    y2 = _tc_mid(m1, y1, dinv, b1.reshape(1, H), g1.reshape(1, H),
                 be1.reshape(1, H), W2)
    m2 = _sc_msg(y2.reshape(NC * NP, HB), src4, dst3).reshape(NC, N, HB)
    y3 = _tc_mid(m2, y2, dinv, b2.reshape(1, H), g2.reshape(1, H),
                 be2.reshape(1, H), W3)
    m3 = _sc_msg(y3.reshape(NC * NP, HB), src4, dst3).reshape(NC, N, HB)
    return _tc_head(m3, y3, dinv, b3.reshape(1, H), batch.reshape(1, N),
                    actions.reshape(1, A), fc1_W[:H], fc1_W[H:],
                    fc1_b.reshape(1, H), fc2_W, fc2_b.reshape(1, H // 2),
                    fc3_W, fc3_b.reshape(1, 1), g3.reshape(1, H),
                    be3.reshape(1, H))
